# baseline (speedup 1.0000x reference)
import functools

import jax
import jax.numpy as jnp
from jax.experimental import pallas as pl


def _relation_kernel(person_ref, other_ref, w_ref, b_ref, out_ref, *,
                     frames, n_p, n_o, d):
    wp = w_ref[:, :d]
    wo = w_ref[:, d:]
    a = jax.lax.dot_general(person_ref[:], wp, (((1,), (1,)), ((), ())),
                            preferred_element_type=jnp.float32)
    b_mat = jax.lax.dot_general(other_ref[:], wo, (((1,), (1,)), ((), ())),
                                preferred_element_type=jnp.float32)
    b_max = jnp.max(b_mat.reshape(frames, n_o, d), axis=1)
    b_rep = jnp.broadcast_to(b_max[:, None, :], (frames, n_p, d))
    out_ref[:] = a + b_rep.reshape(frames * n_p, d) + b_ref[:]


def kernel(person_features, other_features, person_boxes, other_boxes,
           is_person, W, b):
    f_num, n_p = person_boxes.shape[0], person_boxes.shape[1]
    n_o = other_boxes.shape[1]
    d = person_features.shape[1]
    person = person_features.reshape(f_num * n_p, d)
    other = other_features.reshape(f_num * n_o, d)

    frames_per_step = 4 if f_num % 4 == 0 else 1
    steps = f_num // frames_per_step

    out = pl.pallas_call(
        functools.partial(_relation_kernel, frames=frames_per_step,
                          n_p=n_p, n_o=n_o, d=d),
        grid=(steps,),
        in_specs=[
            pl.BlockSpec((frames_per_step * n_p, d), lambda i: (i, 0)),
            pl.BlockSpec((frames_per_step * n_o, d), lambda i: (i, 0)),
            pl.BlockSpec((d, 2 * d), lambda i: (0, 0)),
            pl.BlockSpec((1, d), lambda i: (0, 0)),
        ],
        out_specs=pl.BlockSpec((frames_per_step * n_p, d), lambda i: (i, 0)),
        out_shape=jax.ShapeDtypeStruct((f_num * n_p, d), jnp.float32),
    )(person, other, W, b.reshape(1, d))
    return out[:, :, None, None]


# speedup vs baseline: 1.1151x; 1.1151x over previous
import functools

import jax
import jax.numpy as jnp
from jax.experimental import pallas as pl


def _relation_kernel(person_ref, other_ref, w_ref, b_ref, out_ref, *,
                     frames, n_p, n_o, d):
    wp = w_ref[:, :d]
    wo = w_ref[:, d:]
    a = jax.lax.dot_general(person_ref[:], wp, (((1,), (1,)), ((), ())),
                            preferred_element_type=jnp.float32)
    b_mat = jax.lax.dot_general(other_ref[:], wo, (((1,), (1,)), ((), ())),
                                preferred_element_type=jnp.float32)
    b_max = jnp.max(b_mat.reshape(frames, n_o, d), axis=1)
    b_rep = jnp.broadcast_to(b_max[:, None, :], (frames, n_p, d))
    out_ref[:] = a + b_rep.reshape(frames * n_p, d) + b_ref[:]


def kernel(person_features, other_features, person_boxes, other_boxes,
           is_person, W, b):
    f_num, n_p = person_boxes.shape[0], person_boxes.shape[1]
    n_o = other_boxes.shape[1]
    d = person_features.shape[1]
    person = person_features.reshape(f_num * n_p, d)
    other = other_features.reshape(f_num * n_o, d)

    frames_per_step = 8 if f_num % 8 == 0 else 1
    steps = f_num // frames_per_step

    out = pl.pallas_call(
        functools.partial(_relation_kernel, frames=frames_per_step,
                          n_p=n_p, n_o=n_o, d=d),
        grid=(steps,),
        in_specs=[
            pl.BlockSpec((frames_per_step * n_p, d), lambda i: (i, 0)),
            pl.BlockSpec((frames_per_step * n_o, d), lambda i: (i, 0)),
            pl.BlockSpec((d, 2 * d), lambda i: (0, 0)),
            pl.BlockSpec((1, d), lambda i: (0, 0)),
        ],
        out_specs=pl.BlockSpec((frames_per_step * n_p, d), lambda i: (i, 0)),
        out_shape=jax.ShapeDtypeStruct((f_num * n_p, d), jnp.float32),
    )(person, other, W, b.reshape(1, d))
    return out[:, :, None, None]
